# baseline (speedup 1.0000x reference)
import functools
import math

import jax
import jax.numpy as jnp
from jax import lax
from jax.experimental import pallas as pl
from jax.experimental.pallas import tpu as pltpu
from jax.experimental.pallas import tpu_sc as plsc

ALPHA = 0.99
EPS = 1e-06
LOG_ALPHA = math.log(ALPHA)
M = 100000
D = 64
B = 16384

NC = 2
NS = 16
NW = NC * NS
BPW = B // NW
NCHUNK = 4
CLEN = BPW // NCHUNK

_mesh = plsc.VectorSubcoreMesh(core_axis_name="c", subcore_axis_name="s")


@functools.partial(
    pl.kernel,
    mesh=_mesh,
    out_type=jax.ShapeDtypeStruct((B, D), jnp.float32),
    scratch_types=[
        pltpu.VMEM((NCHUNK, CLEN), jnp.int32),
        pltpu.VMEM((BPW, D), jnp.float32),
        pltpu.VMEM((BPW,), jnp.float32),
        pltpu.SemaphoreType.DMA,
        pltpu.SemaphoreType.DMA,
    ],
    compiler_params=pltpu.CompilerParams(use_tc_tiling_on_sc=False),
)
def _ema_sc(i_hbm, centers_hbm, counts_hbm, out_hbm, idx_v, rows_v, cnt_v,
            sem_rows, sem_cnt):
    wid = lax.axis_index("s") * NC + lax.axis_index("c")
    base = wid * BPW

    for j in range(NCHUNK):
        pltpu.sync_copy(i_hbm.at[pl.ds(base + j * CLEN, CLEN)], idx_v.at[j])

    row_copies = []
    cnt_copies = []
    for j in range(NCHUNK):
        row_copies.append(
            pltpu.async_copy(
                centers_hbm.at[idx_v.at[j]],
                rows_v.at[pl.ds(j * CLEN, CLEN)],
                sem_rows,
            )
        )
        cnt_copies.append(
            pltpu.async_copy(
                counts_hbm.at[idx_v.at[j]],
                cnt_v.at[pl.ds(j * CLEN, CLEN)],
                sem_cnt,
            )
        )
    for c in row_copies:
        c.wait()
    for c in cnt_copies:
        c.wait()

    def body(g, carry):
        cnt16 = cnt_v[pl.ds(g * 16, 16)]
        scale16 = 1.0 / (1.0 + EPS - jnp.exp(cnt16 * LOG_ALPHA))
        for rr in range(16):
            s = lax.gather(
                scale16,
                jnp.full((16, 1), rr, jnp.int32),
                lax.GatherDimensionNumbers(
                    offset_dims=(), collapsed_slice_dims=(0,),
                    start_index_map=(0,)),
                slice_sizes=(1,),
                mode=lax.GatherScatterMode.PROMISE_IN_BOUNDS)
            row = g * 16 + rr
            for c in range(D // 16):
                sl = pl.ds(c * 16, 16)
                rows_v[row, sl] = rows_v[row, sl] * s
        return carry

    lax.fori_loop(0, BPW // 16, body, 0)

    pltpu.sync_copy(rows_v, out_hbm.at[pl.ds(base, BPW)])


def kernel(i, x, centers, counts):
    del x
    return _ema_sc(i.astype(jnp.int32), centers, counts)


# speedup vs baseline: 1.1718x; 1.1718x over previous
import functools
import math

import jax
import jax.numpy as jnp
from jax import lax
from jax.experimental import pallas as pl
from jax.experimental.pallas import tpu as pltpu
from jax.experimental.pallas import tpu_sc as plsc

ALPHA = 0.99
EPS = 1e-06
LOG_ALPHA = math.log(ALPHA)
M = 100000
D = 64
B = 16384

NC = 2
NS = 16
NW = NC * NS
BPW = B // NW
NCHUNK = 4
CLEN = BPW // NCHUNK

TW = 2048


def _tr_body(ct_ref, out_ref):
    yt = lax.transpose(ct_ref[...], (1, 0))
    out_ref[...] = jnp.concatenate([yt, yt], axis=1)


_transpose = pl.pallas_call(
    _tr_body,
    grid=(pl.cdiv(M, TW),),
    in_specs=[pl.BlockSpec((D, TW), lambda j: (0, j))],
    out_specs=pl.BlockSpec((TW, 2 * D), lambda j: (j, 0)),
    out_shape=jax.ShapeDtypeStruct((M, 2 * D), jnp.float32),
)

_mesh = plsc.VectorSubcoreMesh(core_axis_name="c", subcore_axis_name="s")


@functools.partial(
    pl.kernel,
    mesh=_mesh,
    out_type=jax.ShapeDtypeStruct((B, D), jnp.float32),
    scratch_types=[
        pltpu.VMEM((NCHUNK, CLEN), jnp.int32),
        pltpu.VMEM((2, CLEN, 2 * D), jnp.float32),
        pltpu.VMEM((2, CLEN, D), jnp.float32),
        pltpu.VMEM((NCHUNK, CLEN), jnp.float32),
        pltpu.SemaphoreType.DMA,
        pltpu.SemaphoreType.DMA,
        pltpu.SemaphoreType.DMA,
        pltpu.SemaphoreType.DMA,
    ],
)
def _ema_sc(i_hbm, table_hbm, counts_hbm, out_hbm, idx_v, rows_v, out_v,
            cnt_v, sem_g0, sem_g1, sem_cnt, sem_w):
    wid = lax.axis_index("s") * NC + lax.axis_index("c")
    base = wid * BPW
    gsems = (sem_g0, sem_g1)

    for j in range(NCHUNK):
        pltpu.sync_copy(i_hbm.at[pl.ds(base + j * CLEN, CLEN)], idx_v.at[j])

    cnt_copies = [
        pltpu.async_copy(counts_hbm.at[idx_v.at[j]], cnt_v.at[j], sem_cnt)
        for j in range(NCHUNK)
    ]

    def gather_rows(j):
        return pltpu.async_copy(
            table_hbm.at[idx_v.at[j]], rows_v.at[j % 2], gsems[j % 2])

    pending_g = {0: gather_rows(0)}
    pending_w = {}
    for c in cnt_copies:
        c.wait()

    for j in range(NCHUNK):
        if j + 1 < NCHUNK:
            pending_g[j + 1] = gather_rows(j + 1)
        pending_g.pop(j).wait()
        if j - 2 in pending_w:
            pending_w.pop(j - 2).wait()
        b = j % 2

        def body(g, carry):
            cnt16 = cnt_v[j, pl.ds(g * 16, 16)]
            scale16 = 1.0 / (1.0 + EPS - jnp.exp(cnt16 * LOG_ALPHA))
            for rr in range(16):
                s = lax.gather(
                    scale16,
                    jnp.full((16, 1), rr, jnp.int32),
                    lax.GatherDimensionNumbers(
                        offset_dims=(), collapsed_slice_dims=(0,),
                        start_index_map=(0,)),
                    slice_sizes=(1,),
                    mode=lax.GatherScatterMode.PROMISE_IN_BOUNDS)
                row = g * 16 + rr
                for c in range(D // 16):
                    sl = pl.ds(c * 16, 16)
                    out_v[b, row, sl] = rows_v[b, row, sl] * s
            return carry

        lax.fori_loop(0, CLEN // 16, body, 0)
        pending_w[j] = pltpu.async_copy(
            out_v.at[b], out_hbm.at[pl.ds(base + j * CLEN, CLEN)], sem_w)

    for j in sorted(pending_w):
        pending_w.pop(j).wait()


def kernel(i, x, centers, counts):
    del x
    table = _transpose(centers.T)
    return _ema_sc(i.astype(jnp.int32), table, counts)


# speedup vs baseline: 1.2111x; 1.0336x over previous
import functools
import math

import jax
import jax.numpy as jnp
from jax import lax
from jax.experimental import pallas as pl
from jax.experimental.pallas import tpu as pltpu
from jax.experimental.pallas import tpu_sc as plsc

ALPHA = 0.99
EPS = 1e-06
LOG_ALPHA = math.log(ALPHA)
M = 100000
D = 64
B = 16384

NC = 2
NS = 16
NW = NC * NS
BPW = B // NW
NCHUNK = 4
CLEN = BPW // NCHUNK

TW = 2048


def _tr_body(ct_ref, out_ref):
    x = ct_ref[...]
    eye2 = jnp.concatenate(
        [jnp.eye(D, dtype=jnp.float32), jnp.eye(D, dtype=jnp.float32)],
        axis=1)
    out_ref[...] = lax.dot_general(
        x, eye2, (((0,), (0,)), ((), ())),
        preferred_element_type=jnp.float32)


_transpose = pl.pallas_call(
    _tr_body,
    grid=(pl.cdiv(M, TW),),
    in_specs=[pl.BlockSpec((D, TW), lambda j: (0, j))],
    out_specs=pl.BlockSpec((TW, 2 * D), lambda j: (j, 0)),
    out_shape=jax.ShapeDtypeStruct((M, 2 * D), jnp.float32),
)

_mesh = plsc.VectorSubcoreMesh(core_axis_name="c", subcore_axis_name="s")


@functools.partial(
    pl.kernel,
    mesh=_mesh,
    out_type=jax.ShapeDtypeStruct((B, D), jnp.float32),
    scratch_types=[
        pltpu.VMEM((NCHUNK, CLEN), jnp.int32),
        pltpu.VMEM((2, CLEN, 2 * D), jnp.float32),
        pltpu.VMEM((2, CLEN, D), jnp.float32),
        pltpu.VMEM((NCHUNK, CLEN), jnp.float32),
        pltpu.SemaphoreType.DMA,
        pltpu.SemaphoreType.DMA,
        pltpu.SemaphoreType.DMA,
        pltpu.SemaphoreType.DMA,
    ],
)
def _ema_sc(i_hbm, table_hbm, counts_hbm, out_hbm, idx_v, rows_v, out_v,
            cnt_v, sem_g0, sem_g1, sem_cnt, sem_w):
    wid = lax.axis_index("s") * NC + lax.axis_index("c")
    base = wid * BPW
    gsems = (sem_g0, sem_g1)

    for j in range(NCHUNK):
        pltpu.sync_copy(i_hbm.at[pl.ds(base + j * CLEN, CLEN)], idx_v.at[j])

    cnt_copies = [
        pltpu.async_copy(counts_hbm.at[idx_v.at[j]], cnt_v.at[j], sem_cnt)
        for j in range(NCHUNK)
    ]

    def gather_rows(j):
        return pltpu.async_copy(
            table_hbm.at[idx_v.at[j]], rows_v.at[j % 2], gsems[j % 2])

    pending_g = {0: gather_rows(0)}
    pending_w = {}
    for c in cnt_copies:
        c.wait()

    for j in range(NCHUNK):
        if j + 1 < NCHUNK:
            pending_g[j + 1] = gather_rows(j + 1)
        pending_g.pop(j).wait()
        if j - 2 in pending_w:
            pending_w.pop(j - 2).wait()
        b = j % 2

        def body(g, carry):
            cnt16 = cnt_v[j, pl.ds(g * 16, 16)]
            scale16 = 1.0 / (1.0 + EPS - jnp.exp(cnt16 * LOG_ALPHA))
            for rr in range(16):
                s = lax.gather(
                    scale16,
                    jnp.full((16, 1), rr, jnp.int32),
                    lax.GatherDimensionNumbers(
                        offset_dims=(), collapsed_slice_dims=(0,),
                        start_index_map=(0,)),
                    slice_sizes=(1,),
                    mode=lax.GatherScatterMode.PROMISE_IN_BOUNDS)
                row = g * 16 + rr
                for c in range(D // 16):
                    sl = pl.ds(c * 16, 16)
                    out_v[b, row, sl] = rows_v[b, row, sl] * s
            return carry

        lax.fori_loop(0, CLEN // 16, body, 0)
        pending_w[j] = pltpu.async_copy(
            out_v.at[b], out_hbm.at[pl.ds(base + j * CLEN, CLEN)], sem_w)

    for j in sorted(pending_w):
        pending_w.pop(j).wait()


def kernel(i, x, centers, counts):
    del x
    table = _transpose(centers.T)
    return _ema_sc(i.astype(jnp.int32), table, counts)


# speedup vs baseline: 1.5217x; 1.2564x over previous
import functools
import math

import jax
import jax.numpy as jnp
from jax import lax
from jax.experimental import pallas as pl
from jax.experimental.pallas import tpu as pltpu
from jax.experimental.pallas import tpu_sc as plsc

ALPHA = 0.99
EPS = 1e-06
LOG_ALPHA = math.log(ALPHA)
M = 100000
D = 64
B = 16384

NC = 2
NS = 16
NW = NC * NS
FPW = D // NW
BROWS = B // 128
P0R = BROWS // NS
NBCH = 4
CROWS = BROWS // NBCH
CB = CROWS * 128

_mesh = plsc.VectorSubcoreMesh(core_axis_name="c", subcore_axis_name="s")


@functools.partial(
    pl.kernel,
    mesh=_mesh,
    out_type=jax.ShapeDtypeStruct((D, B), jnp.float32),
    compiler_params=pltpu.CompilerParams(needs_layout_passes=False),
    scratch_types=[
        pltpu.VMEM((P0R, 128), jnp.int32),
        pltpu.VMEM((P0R, 128), jnp.float32),
        pltpu.VMEM((P0R, 128), jnp.float32),
        pltpu.VMEM_SHARED((BROWS, 128), jnp.float32),
        pltpu.VMEM((M,), jnp.float32),
        pltpu.VMEM((CROWS, 128), jnp.int32),
        pltpu.VMEM((CROWS, 128), jnp.float32),
        pltpu.VMEM((2, CB), jnp.float32),
        pltpu.SemaphoreType.DMA,
        pltpu.SemaphoreType.DMA,
    ],
)
def _ema_sc(i_hbm, ct_hbm, counts_hbm, out_hbm, idx0_v, cnt0_v, scale0_v,
            scale_sh, row_v, idxc_v, scalec_v, outc_v, sem_g, sem_w):
    cid = lax.axis_index("c")
    sid = lax.axis_index("s")
    wid = sid * NC + cid

    pltpu.sync_copy(i_hbm.at[pl.ds(sid * P0R, P0R)], idx0_v)
    cnt_copies = [
        pltpu.async_copy(counts_hbm.at[idx0_v.at[r]], cnt0_v.at[r], sem_g)
        for r in range(P0R)
    ]
    for c in cnt_copies:
        c.wait()
    for r in range(P0R):
        for c in range(128 // 16):
            sl = pl.ds(c * 16, 16)
            scale0_v[r, sl] = 1.0 / (
                1.0 + EPS - jnp.exp(cnt0_v[r, sl] * LOG_ALPHA))
    pltpu.sync_copy(scale0_v, scale_sh.at[pl.ds(sid * P0R, P0R)])
    plsc.subcore_barrier()

    pending_w = {}
    for f in range(FPW):
        d = wid * FPW + f
        pltpu.sync_copy(ct_hbm.at[d], row_v)
        for k in range(NBCH):
            pltpu.sync_copy(i_hbm.at[pl.ds(k * CROWS, CROWS)], idxc_v)
            pltpu.sync_copy(scale_sh.at[pl.ds(k * CROWS, CROWS)], scalec_v)
            if (f, k - 2) in pending_w:
                pending_w.pop((f, k - 2)).wait()
            if (f - 1, k + 2) in pending_w:
                pending_w.pop((f - 1, k + 2)).wait()
            bb = k % 2

            def body(rr, carry):
                for c in range(128 // 16):
                    sl = pl.ds(c * 16, 16)
                    idx16 = idxc_v[rr, sl]
                    v16 = plsc.load_gather(row_v, [idx16])
                    outc_v[bb, pl.ds(rr * 128 + c * 16, 16)] = (
                        v16 * scalec_v[rr, sl])
                return carry

            lax.fori_loop(0, CROWS, body, 0)
            pending_w[(f, k)] = pltpu.async_copy(
                outc_v.at[bb], out_hbm.at[d, pl.ds(k * CB, CB)], sem_w)
    for key in sorted(pending_w):
        pending_w.pop(key).wait()


def kernel(i, x, centers, counts):
    del x
    i2d = i.astype(jnp.int32).reshape(BROWS, 128)
    out_t = _ema_sc(i2d, centers.T, counts)
    return out_t.T


# speedup vs baseline: 1.9297x; 1.2682x over previous
import functools
import math

import jax
import jax.numpy as jnp
from jax import lax
from jax.experimental import pallas as pl
from jax.experimental.pallas import tpu as pltpu
from jax.experimental.pallas import tpu_sc as plsc

ALPHA = 0.99
EPS = 1e-06
LOG_ALPHA = math.log(ALPHA)
M = 100000
D = 64
B = 16384

NC = 2
NS = 16
NW = NC * NS
FPW = D // NW
BROWS = B // 128
P0R = BROWS // NS
NBCH = 4
CROWS = BROWS // NBCH
CB = CROWS * 128

_mesh = plsc.VectorSubcoreMesh(core_axis_name="c", subcore_axis_name="s")


@functools.partial(
    pl.kernel,
    mesh=_mesh,
    out_type=jax.ShapeDtypeStruct((D, B), jnp.float32),
    compiler_params=pltpu.CompilerParams(needs_layout_passes=False),
    scratch_types=[
        pltpu.VMEM((P0R, 128), jnp.int32),
        pltpu.VMEM((P0R, 128), jnp.float32),
        pltpu.VMEM((P0R, 128), jnp.float32),
        pltpu.VMEM_SHARED((BROWS, 128), jnp.float32),
        pltpu.VMEM((M,), jnp.float32),
        pltpu.VMEM((CROWS, 128), jnp.int32),
        pltpu.VMEM((CROWS, 128), jnp.float32),
        pltpu.VMEM((2, CB), jnp.float32),
        pltpu.SemaphoreType.DMA,
        pltpu.SemaphoreType.DMA,
    ],
)
def _ema_sc(i_hbm, ct_hbm, counts_hbm, out_hbm, idx0_v, cnt0_v, scale0_v,
            scale_sh, row_v, idxc_v, scalec_v, outc_v, sem_g, sem_w):
    cid = lax.axis_index("c")
    sid = lax.axis_index("s")
    wid = sid * NC + cid

    pltpu.sync_copy(i_hbm.at[pl.ds(sid * P0R, P0R)], idx0_v)
    cnt_copies = [
        pltpu.async_copy(counts_hbm.at[idx0_v.at[r]], cnt0_v.at[r], sem_g)
        for r in range(P0R)
    ]
    for c in cnt_copies:
        c.wait()
    for r in range(P0R):
        for c in range(128 // 16):
            sl = pl.ds(c * 16, 16)
            scale0_v[r, sl] = 1.0 / (
                1.0 + EPS - jnp.exp(cnt0_v[r, sl] * LOG_ALPHA))
    pltpu.sync_copy(scale0_v, scale_sh.at[pl.ds(sid * P0R, P0R)])
    plsc.subcore_barrier()

    pending_w = {}
    for f in range(FPW):
        d = wid * FPW + f
        pltpu.sync_copy(ct_hbm.at[d], row_v)
        for k in range(NBCH):
            pltpu.sync_copy(i_hbm.at[pl.ds(k * CROWS, CROWS)], idxc_v)
            pltpu.sync_copy(scale_sh.at[pl.ds(k * CROWS, CROWS)], scalec_v)
            if (f, k - 2) in pending_w:
                pending_w.pop((f, k - 2)).wait()
            if (f - 1, k + 2) in pending_w:
                pending_w.pop((f - 1, k + 2)).wait()
            bb = k % 2

            def body(rr, carry):
                sls = [pl.ds(c * 16, 16) for c in range(128 // 16)]
                idxs = [idxc_v[rr, sl] for sl in sls]
                gath = [plsc.load_gather(row_v, [ix]) for ix in idxs]
                scls = [scalec_v[rr, sl] for sl in sls]
                for c in range(128 // 16):
                    outc_v[bb, pl.ds(rr * 128 + c * 16, 16)] = (
                        gath[c] * scls[c])
                return carry

            lax.fori_loop(0, CROWS, body, 0)
            pending_w[(f, k)] = pltpu.async_copy(
                outc_v.at[bb], out_hbm.at[d, pl.ds(k * CB, CB)], sem_w)
    for key in sorted(pending_w):
        pending_w.pop(key).wait()


def kernel(i, x, centers, counts):
    del x
    i2d = i.astype(jnp.int32).reshape(BROWS, 128)
    out_t = _ema_sc(i2d, centers.T, counts)
    return out_t.T


# speedup vs baseline: 2.2718x; 1.1772x over previous
import functools
import math

import jax
import jax.numpy as jnp
from jax import lax
from jax.experimental import pallas as pl
from jax.experimental.pallas import tpu as pltpu
from jax.experimental.pallas import tpu_sc as plsc

ALPHA = 0.99
EPS = 1e-06
LOG_ALPHA = math.log(ALPHA)
M = 100000
D = 64
B = 16384

NC = 2
NS = 16
NW = NC * NS
FPW = D // NW
BROWS = B // 128
P0R = BROWS // NS
NBCH = 4
CROWS = BROWS // NBCH
CB = CROWS * 128

_mesh = plsc.VectorSubcoreMesh(core_axis_name="c", subcore_axis_name="s")


@functools.partial(
    pl.kernel,
    mesh=_mesh,
    out_type=jax.ShapeDtypeStruct((D, B), jnp.float32),
    compiler_params=pltpu.CompilerParams(needs_layout_passes=False),
    scratch_types=[
        pltpu.VMEM((P0R, 128), jnp.int32),
        pltpu.VMEM((P0R, 128), jnp.float32),
        pltpu.VMEM((P0R, 128), jnp.float32),
        pltpu.VMEM_SHARED((BROWS, 128), jnp.float32),
        pltpu.VMEM((M,), jnp.float32),
        pltpu.VMEM((2, CROWS, 128), jnp.int32),
        pltpu.VMEM((2, CROWS, 128), jnp.float32),
        pltpu.VMEM((2, CB), jnp.float32),
        pltpu.SemaphoreType.DMA,
        pltpu.SemaphoreType.DMA,
        pltpu.SemaphoreType.DMA,
        pltpu.SemaphoreType.DMA,
        pltpu.SemaphoreType.DMA,
    ],
)
def _ema_sc(i_hbm, ct_hbm, counts_hbm, out_hbm, idx0_v, cnt0_v, scale0_v,
            scale_sh, row_v, idxc_v, scalec_v, outc_v, sem_g, sem_w,
            sem_r, sem_p0, sem_p1):
    cid = lax.axis_index("c")
    sid = lax.axis_index("s")
    wid = sid * NC + cid

    row_cp = pltpu.async_copy(ct_hbm.at[wid * FPW], row_v, sem_r)

    pltpu.sync_copy(i_hbm.at[pl.ds(sid * P0R, P0R)], idx0_v)
    cnt_copies = [
        pltpu.async_copy(counts_hbm.at[idx0_v.at[r]], cnt0_v.at[r], sem_g)
        for r in range(P0R)
    ]
    for c in cnt_copies:
        c.wait()
    for r in range(P0R):
        for c in range(128 // 16):
            sl = pl.ds(c * 16, 16)
            scale0_v[r, sl] = 1.0 / (
                1.0 + EPS - jnp.exp(cnt0_v[r, sl] * LOG_ALPHA))
    pltpu.sync_copy(scale0_v, scale_sh.at[pl.ds(sid * P0R, P0R)])
    plsc.subcore_barrier()

    psems = (sem_p0, sem_p1)

    def prefetch(s):
        k = s % NBCH
        pb = s % 2
        return [
            pltpu.async_copy(
                i_hbm.at[pl.ds(k * CROWS, CROWS)], idxc_v.at[pb],
                psems[pb]),
        ]

    nsteps = FPW * NBCH
    pending_p = {0: prefetch(0)}
    pending_w = {}
    for s in range(nsteps):
        f, k = s // NBCH, s % NBCH
        if k == 0:
            if f == 0:
                row_cp.wait()
            else:
                pltpu.sync_copy(ct_hbm.at[wid * FPW + f], row_v)
        for c in pending_p.pop(s):
            c.wait()
        pltpu.sync_copy(scale_sh.at[pl.ds(k * CROWS, CROWS)],
                        scalec_v.at[s % 2])
        if s + 1 < nsteps:
            pending_p[s + 1] = prefetch(s + 1)
        if s - 2 in pending_w:
            pending_w.pop(s - 2).wait()
        bb = s % 2
        d = wid * FPW + f

        def body(rr, carry):
            sls = [pl.ds(c * 16, 16) for c in range(128 // 16)]
            idxs = [idxc_v[bb, rr, sl] for sl in sls]
            gath = [plsc.load_gather(row_v, [ix]) for ix in idxs]
            scls = [scalec_v[bb, rr, sl] for sl in sls]
            for c in range(128 // 16):
                outc_v[bb, pl.ds(rr * 128 + c * 16, 16)] = (
                    gath[c] * scls[c])
            return carry

        lax.fori_loop(0, CROWS, body, 0)
        pending_w[s] = pltpu.async_copy(
            outc_v.at[bb], out_hbm.at[d, pl.ds(k * CB, CB)], sem_w)
    for key in sorted(pending_w):
        pending_w.pop(key).wait()


def kernel(i, x, centers, counts):
    del x
    i2d = i.astype(jnp.int32).reshape(BROWS, 128)
    out_t = _ema_sc(i2d, centers.T, counts)
    return out_t.T
